# baseline (speedup 1.0000x reference)
import functools
import jax
import jax.numpy as jnp
from jax import lax
from jax.experimental import pallas as pl
from jax.experimental.pallas import tpu as pltpu
from jax.experimental.pallas import tpu_sc as plsc

BATCH = 16384
DIM = 64

_info = plsc.get_sparse_core_info()
_NC, _NS = _info.num_cores, _info.num_subcores
_NW = _NC * _NS
_BPW = BATCH // _NW
_NCHUNK = 2
_C = _BPW // _NCHUNK


def _sc_gather(h, r, t, ent_emb, rel_emb, rel_norm):
    mesh = plsc.VectorSubcoreMesh(core_axis_name="c", subcore_axis_name="s")
    out = jax.ShapeDtypeStruct((BATCH, DIM), jnp.float32)

    @functools.partial(
        pl.kernel,
        mesh=mesh,
        out_type=[out, out, out, out],
        compiler_params=pltpu.CompilerParams(use_tc_tiling_on_sc=False),
        scratch_types=(
            [pltpu.VMEM((_C,), jnp.int32) for _ in range(3 * _NCHUNK)]
            + [
                pltpu.VMEM((_C, DIM), jnp.float32),
                pltpu.VMEM((_C, DIM), jnp.float32),
                pltpu.VMEM((_C, DIM), jnp.float32),
                pltpu.VMEM((_C, DIM), jnp.float32),
                pltpu.SemaphoreType.DMA,
            ]
        ),
    )
    def k(h_hbm, r_hbm, t_hbm, ent_hbm, rele_hbm, reln_hbm,
          ho_hbm, ro_hbm, to_hbm, wo_hbm,
          *scratch):
        idx = scratch[: 3 * _NCHUNK]
        hb, rb, tb, wb, sem = scratch[3 * _NCHUNK:]
        wid = lax.axis_index("s") * _NC + lax.axis_index("c")
        base = wid * _BPW
        for c in range(_NCHUNK):
            off = base + c * _C
            ih, ir, it = idx[3 * c: 3 * c + 3]
            pltpu.sync_copy(h_hbm.at[pl.ds(off, _C)], ih)
            pltpu.sync_copy(r_hbm.at[pl.ds(off, _C)], ir)
            pltpu.sync_copy(t_hbm.at[pl.ds(off, _C)], it)
            cps = [
                pltpu.async_copy(ent_hbm.at[ih], hb, sem),
                pltpu.async_copy(rele_hbm.at[ir], rb, sem),
                pltpu.async_copy(ent_hbm.at[it], tb, sem),
                pltpu.async_copy(reln_hbm.at[ir], wb, sem),
            ]
            for cp in cps:
                cp.wait()
            pltpu.sync_copy(hb, ho_hbm.at[pl.ds(off, _C)])
            pltpu.sync_copy(rb, ro_hbm.at[pl.ds(off, _C)])
            pltpu.sync_copy(tb, to_hbm.at[pl.ds(off, _C)])
            pltpu.sync_copy(wb, wo_hbm.at[pl.ds(off, _C)])

    return k(h, r, t, ent_emb, rel_emb, rel_norm)


def _tc_score_body(h_ref, r_ref, t_ref, w_ref, o_ref):
    w = w_ref[...]
    d = h_ref[...] + r_ref[...] - t_ref[...]
    m = jnp.maximum(jnp.sqrt(jnp.sum(w * w, axis=1, keepdims=True)), 1e-12)
    wn = w / m
    s1 = jnp.sum(d * wn, axis=1, keepdims=True)
    proj = d - s1 * wn
    o_ref[...] = jnp.sqrt(jnp.sum(proj * proj, axis=1))


def _tc_score(h_rows, r_rows, t_rows, w_rows):
    return pl.pallas_call(
        _tc_score_body,
        out_shape=jax.ShapeDtypeStruct((BATCH,), jnp.float32),
    )(h_rows, r_rows, t_rows, w_rows)


@jax.jit
def kernel(h, r, t, ent_emb, rel_emb, rel_norm):
    h_rows, r_rows, t_rows, w_rows = _sc_gather(h, r, t, ent_emb, rel_emb, rel_norm)
    return _tc_score(h_rows, r_rows, t_rows, w_rows)
